# baseline (speedup 1.0000x reference)
import jax
import jax.numpy as jnp
from jax.experimental import pallas as pl
from jax.experimental.pallas import tpu as pltpu

_C = 80
_N = 20000
_K = 300
_FLAT = _N * _C
_LANES = 128
_ROWS = 12500
_CHUNK = 128
_NCHUNK = 98
_PAD_ROWS = _NCHUNK * _CHUNK
_BIG = 1 << 30


def _topk_body(size_ref, score_ref, box_ref, out_s_ref, out_l_ref, out_b_ref,
               s_ref, m_ref):
    b = pl.program_id(0)
    x = score_ref[0]
    rowid = (jax.lax.broadcasted_iota(jnp.int32, x.shape, 0) * _CHUNK
             + jax.lax.broadcasted_iota(jnp.int32, x.shape, 1))
    x = jnp.where(rowid < _ROWS, x, -1.0)
    s_ref[...] = x
    m_ref[...] = jnp.max(x, axis=2)

    h = size_ref[b, 0].astype(jnp.float32)
    w = size_ref[b, 1].astype(jnp.float32)

    lane = jax.lax.broadcasted_iota(jnp.int32, (1, _LANES), 1)

    def init_body(i, m1):
        return jnp.where(lane == i, jnp.max(m_ref[pl.ds(i, 1), :]), m1)

    m1 = jax.lax.fori_loop(0, _NCHUNK, init_body,
                           jnp.full((1, _LANES), -1.0, jnp.float32))

    def body(t, m1):
        v = jnp.max(m1)
        i = jnp.min(jnp.where(m1 == v, lane, _BIG))
        mrow = m_ref[pl.ds(i, 1), :]
        j = jnp.min(jnp.where(mrow == v, lane, _BIG))
        row = s_ref[i, pl.ds(j, 1), :]
        k = jnp.min(jnp.where(row == v, lane, _BIG))
        e = (i * _CHUNK + j) * _LANES + k

        new_row = jnp.where(lane == k, -1.0, row)
        s_ref[i, pl.ds(j, 1), :] = new_row
        new_mrow = jnp.where(lane == j, jnp.max(new_row), mrow)
        m_ref[pl.ds(i, 1), :] = new_mrow
        m1 = jnp.where(lane == i, jnp.max(new_mrow), m1)

        label = e % _C
        q = e // _C
        out_s_ref[b, t] = v
        out_l_ref[b, t] = label

        r = q // 32
        o = (q - r * 32) * 4
        brow = box_ref[0, pl.ds(r, 1), :]
        cx = jnp.sum(jnp.where(lane == o, brow, 0.0))
        cy = jnp.sum(jnp.where(lane == o + 1, brow, 0.0))
        bw = jnp.sum(jnp.where(lane == o + 2, brow, 0.0))
        bh = jnp.sum(jnp.where(lane == o + 3, brow, 0.0))
        out_b_ref[b, 4 * t + 0] = (cx - 0.5 * bw) * w
        out_b_ref[b, 4 * t + 1] = (cy - 0.5 * bh) * h
        out_b_ref[b, 4 * t + 2] = (cx + 0.5 * bw) * w
        out_b_ref[b, 4 * t + 3] = (cy + 0.5 * bh) * h
        return m1

    jax.lax.fori_loop(0, _K, body, m1)


def kernel(pred_logits, pred_boxes, orig_target_sizes):
    B = pred_logits.shape[0]
    scores = jax.nn.sigmoid(pred_logits).reshape(B, _FLAT)
    pad = _PAD_ROWS * _LANES - _FLAT
    scores = jnp.pad(scores, ((0, 0), (0, pad)))
    scores = scores.reshape(B, _NCHUNK, _CHUNK, _LANES)
    boxes = pred_boxes.reshape(B, _N // 32, _LANES)

    out_s, out_l, out_b = pl.pallas_call(
        _topk_body,
        grid=(B,),
        in_specs=[
            pl.BlockSpec((8, 2), lambda b: (0, 0), memory_space=pltpu.SMEM),
            pl.BlockSpec((1, _NCHUNK, _CHUNK, _LANES), lambda b: (b, 0, 0, 0)),
            pl.BlockSpec((1, _N // 32, _LANES), lambda b: (b, 0, 0)),
        ],
        out_specs=[
            pl.BlockSpec((8, _K), lambda b: (0, 0), memory_space=pltpu.SMEM),
            pl.BlockSpec((8, _K), lambda b: (0, 0), memory_space=pltpu.SMEM),
            pl.BlockSpec((8, 4 * _K), lambda b: (0, 0), memory_space=pltpu.SMEM),
        ],
        out_shape=[
            jax.ShapeDtypeStruct((B, _K), jnp.float32),
            jax.ShapeDtypeStruct((B, _K), jnp.int32),
            jax.ShapeDtypeStruct((B, 4 * _K), jnp.float32),
        ],
        scratch_shapes=[
            pltpu.VMEM((_NCHUNK, _CHUNK, _LANES), jnp.float32),
            pltpu.VMEM((_NCHUNK, _CHUNK), jnp.float32),
        ],
    )(orig_target_sizes, scores, boxes)

    return out_s, out_l, out_b.reshape(B, _K, 4)


# speedup vs baseline: 1.2273x; 1.2273x over previous
import jax
import jax.numpy as jnp
from jax.experimental import pallas as pl
from jax.experimental.pallas import tpu as pltpu

_C = 80
_N = 20000
_K = 300
_FLAT = _N * _C
_LANES = 128
_ROWS = 12500
_CHUNK = 128
_NCHUNK = 98
_PAD_ROWS = _NCHUNK * _CHUNK
_BIG = 1 << 30
_B = 8


def _topk_body(size_ref, score_ref, box_ref, out_s_ref, out_l_ref, out_b_ref,
               m_ref):
    rowid = (jax.lax.broadcasted_iota(jnp.int32, (_NCHUNK, _CHUNK, _LANES), 0)
             * _CHUNK
             + jax.lax.broadcasted_iota(jnp.int32, (_NCHUNK, _CHUNK, _LANES), 1))
    for b in range(_B):
        x = jnp.where(rowid < _ROWS, score_ref[b], -1.0)
        score_ref[b] = x
        m_ref[b] = jnp.max(x, axis=2)

    hw = [(size_ref[b, 0].astype(jnp.float32),
           size_ref[b, 1].astype(jnp.float32)) for b in range(_B)]

    flat_ij = (jax.lax.broadcasted_iota(jnp.int32, (_NCHUNK, _CHUNK), 0) * _CHUNK
               + jax.lax.broadcasted_iota(jnp.int32, (_NCHUNK, _CHUNK), 1))
    lane = jax.lax.broadcasted_iota(jnp.int32, (1, _LANES), 1)

    def body(t, _):
        for b in range(_B):
            h, w = hw[b]
            m = m_ref[b]
            v = jnp.max(m)
            p = jnp.min(jnp.where(m == v, flat_ij, _BIG))
            i = p // _CHUNK
            j = p - i * _CHUNK
            row = score_ref[b, i, pl.ds(j, 1), :]
            k = jnp.min(jnp.where(row == v, lane, _BIG))
            e = p * _LANES + k

            new_row = jnp.where(lane == k, -1.0, row)
            score_ref[b, i, pl.ds(j, 1), :] = new_row
            mrow = m_ref[b, pl.ds(i, 1), :]
            m_ref[b, pl.ds(i, 1), :] = jnp.where(lane == j, jnp.max(new_row),
                                                 mrow)

            label = e % _C
            q = e // _C
            out_s_ref[b, t] = v
            out_l_ref[b, t] = label

            r = q // 32
            o = (q - r * 32) * 4
            brow = box_ref[b, pl.ds(r, 1), :]
            cx = jnp.sum(jnp.where(lane == o, brow, 0.0))
            cy = jnp.sum(jnp.where(lane == o + 1, brow, 0.0))
            bw = jnp.sum(jnp.where(lane == o + 2, brow, 0.0))
            bh = jnp.sum(jnp.where(lane == o + 3, brow, 0.0))
            out_b_ref[b, 4 * t + 0] = (cx - 0.5 * bw) * w
            out_b_ref[b, 4 * t + 1] = (cy - 0.5 * bh) * h
            out_b_ref[b, 4 * t + 2] = (cx + 0.5 * bw) * w
            out_b_ref[b, 4 * t + 3] = (cy + 0.5 * bh) * h
        return 0

    jax.lax.fori_loop(0, _K, body, 0)


def kernel(pred_logits, pred_boxes, orig_target_sizes):
    B = pred_logits.shape[0]
    scores = jax.nn.sigmoid(pred_logits).reshape(B, _FLAT)
    pad = _PAD_ROWS * _LANES - _FLAT
    scores = jnp.pad(scores, ((0, 0), (0, pad)))
    scores = scores.reshape(B, _NCHUNK, _CHUNK, _LANES)
    boxes = pred_boxes.reshape(B, _N // 32, _LANES)

    out_s, out_l, out_b = pl.pallas_call(
        _topk_body,
        in_specs=[
            pl.BlockSpec(memory_space=pltpu.SMEM),
            pl.BlockSpec((_B, _NCHUNK, _CHUNK, _LANES),
                         lambda: (0, 0, 0, 0)),
            pl.BlockSpec((_B, _N // 32, _LANES), lambda: (0, 0, 0)),
        ],
        out_specs=[
            pl.BlockSpec(memory_space=pltpu.SMEM),
            pl.BlockSpec(memory_space=pltpu.SMEM),
            pl.BlockSpec(memory_space=pltpu.SMEM),
        ],
        out_shape=[
            jax.ShapeDtypeStruct((B, _K), jnp.float32),
            jax.ShapeDtypeStruct((B, _K), jnp.int32),
            jax.ShapeDtypeStruct((B, 4 * _K), jnp.float32),
        ],
        scratch_shapes=[
            pltpu.VMEM((_B, _NCHUNK, _CHUNK), jnp.float32),
        ],
    )(orig_target_sizes, scores, boxes)

    return out_s, out_l, out_b.reshape(B, _K, 4)
